# baseline (speedup 1.0000x reference)
import functools

import jax
import jax.numpy as jnp
from jax.experimental import pallas as pl
from jax.experimental.pallas import tpu as pltpu

_S = 4096
_D = 768
_H = 12
_DEPTH = 64
_W = 8
_T = 512


def _layer_norm_in(x, g, b, eps=1e-3):
    mu = jnp.mean(x, axis=-1, keepdims=True)
    xc = x - mu
    var = jnp.mean(xc * xc, axis=-1, keepdims=True)
    return g * xc * jax.lax.rsqrt(var + eps) + b


def _body(xb_ref, halo_ref, g1_ref, b1_ref, wq_ref, bq_ref, wk_ref, bk_ref,
          wv_ref, bv_ref, g2_ref, b2_ref, wo_ref, bo_ref, out_ref):
    i = pl.program_id(0)
    xfull = jnp.concatenate([halo_ref[...], xb_ref[...]], axis=0)
    xl = _layer_norm_in(xfull, g1_ref[...], b1_ref[...])

    xq = xl[_W:, :]
    xlb = xl.astype(jnp.bfloat16)
    q = jnp.dot(xlb[_W:, :], wq_ref[...], preferred_element_type=jnp.float32) + bq_ref[...]
    k = jnp.dot(xlb, wk_ref[...], preferred_element_type=jnp.float32) + bk_ref[...]
    v = jnp.dot(xlb, wv_ref[...], preferred_element_type=jnp.float32) + bv_ref[...]

    r_idx = jax.lax.broadcasted_iota(jnp.int32, (_T, _T + _W), 0)
    m_idx = jax.lax.broadcasted_iota(jnp.int32, (_T, _T + _W), 1)
    diff = r_idx + _W - m_idx
    valid = (diff >= 0) & (diff < _W) & (m_idx + i * _T >= _W)

    chunks = []
    for h in range(_H):
        sl = slice(h * _DEPTH, (h + 1) * _DEPTH)
        lg = jax.lax.dot_general(q[:, sl], k[:, sl], (((1,), (1,)), ((), ())),
                                 preferred_element_type=jnp.float32)
        e = jnp.where(valid, jnp.exp(lg), 0.0)
        den = jnp.sum(e, axis=1, keepdims=True)
        num = jnp.dot(e, v[:, sl], preferred_element_type=jnp.float32)
        chunks.append(num / den)
    att = jnp.concatenate(chunks, axis=1)

    concat = xq + att
    y = _layer_norm_in(concat, g2_ref[...], b2_ref[...])
    y = jnp.dot(y.astype(jnp.bfloat16), wo_ref[...],
                preferred_element_type=jnp.float32) + bo_ref[...]
    y = jnp.maximum(y, 0.0)
    out_ref[...] = y + concat


@functools.partial(jax.jit, static_argnames=())
def _run(x2, ln1_g, ln1_b, Wq, bq, Wk, bk, Wv, bv, ln2_g, ln2_b, Wo, bo):
    grid = (_S // _T,)
    vec = pl.BlockSpec((1, _D), lambda i: (0, 0))
    mat = pl.BlockSpec((_D, _D), lambda i: (0, 0))
    out = pl.pallas_call(
        _body,
        grid=grid,
        in_specs=[
            pl.BlockSpec((_T, _D), lambda i: (i, 0)),
            pl.BlockSpec((_W, _D),
                         lambda i: (jnp.maximum(i * (_T // _W) - 1, 0), 0)),
            vec, vec, mat, vec, mat, vec, mat, vec, vec, vec, mat, vec,
        ],
        out_specs=pl.BlockSpec((_T, _D), lambda i: (i, 0)),
        out_shape=jax.ShapeDtypeStruct((_S, _D), jnp.float32),
        compiler_params=pltpu.CompilerParams(
            dimension_semantics=("arbitrary",),
        ),
    )(x2, x2, ln1_g, ln1_b, Wq, bq, Wk, bk, Wv, bv, ln2_g, ln2_b, Wo, bo)
    return out


def kernel(inputs, ln1_g, ln1_b, Wq, bq, Wk, bk, Wv, bv, ln2_g, ln2_b, Wo, bo, mask):
    del mask
    x2 = inputs.reshape(_S, _D)
    r = lambda a: a.reshape(1, _D)
    scale = 1.0 / (_DEPTH ** 0.5)
    out = _run(x2, r(ln1_g), r(ln1_b), (Wq * scale).astype(jnp.bfloat16),
               r(bq * scale), Wk.astype(jnp.bfloat16), r(bk),
               Wv.astype(jnp.bfloat16), r(bv),
               r(ln2_g), r(ln2_b), Wo.astype(jnp.bfloat16), r(bo))
    return out.reshape(1, _S, _D)


# speedup vs baseline: 1.0325x; 1.0325x over previous
import functools

import jax
import jax.numpy as jnp
from jax.experimental import pallas as pl
from jax.experimental.pallas import tpu as pltpu

_S = 4096
_D = 768
_H = 12
_DEPTH = 64
_W = 8
_T = 512


def _layer_norm_in(x, g, b, eps=1e-3):
    mu = jnp.mean(x, axis=-1, keepdims=True)
    xc = x - mu
    var = jnp.mean(xc * xc, axis=-1, keepdims=True)
    return g * xc * jax.lax.rsqrt(var + eps) + b


def _body(xb_ref, halo_ref, g1_ref, b1_ref, wq_ref, bq_ref, wk_ref, bk_ref,
          wv_ref, bv_ref, g2_ref, b2_ref, wo_ref, bo_ref, out_ref):
    i = pl.program_id(0)
    xfull = jnp.concatenate([halo_ref[...], xb_ref[...]], axis=0)
    xl = _layer_norm_in(xfull, g1_ref[...], b1_ref[...])

    xq = xl[_W:, :]
    q = jnp.dot(xq, wq_ref[...], preferred_element_type=jnp.float32) + bq_ref[...]
    k = jnp.dot(xl, wk_ref[...], preferred_element_type=jnp.float32) + bk_ref[...]
    v = jnp.dot(xl, wv_ref[...], preferred_element_type=jnp.float32) + bv_ref[...]

    r_idx = jax.lax.broadcasted_iota(jnp.int32, (_T, _T + _W), 0)
    m_idx = jax.lax.broadcasted_iota(jnp.int32, (_T, _T + _W), 1)
    diff = r_idx + _W - m_idx
    valid = (diff >= 0) & (diff < _W) & (m_idx + i * _T >= _W)

    chunks = []
    for h in range(_H):
        sl = slice(h * _DEPTH, (h + 1) * _DEPTH)
        lg = jax.lax.dot_general(q[:, sl], k[:, sl], (((1,), (1,)), ((), ())),
                                 preferred_element_type=jnp.float32)
        e = jnp.where(valid, jnp.exp(lg), 0.0)
        den = jnp.sum(e, axis=1, keepdims=True)
        num = jnp.dot(e, v[:, sl], preferred_element_type=jnp.float32)
        chunks.append(num / den)
    att = jnp.concatenate(chunks, axis=1)

    concat = xq + att
    y = _layer_norm_in(concat, g2_ref[...], b2_ref[...])
    y = jnp.dot(y, wo_ref[...], preferred_element_type=jnp.float32) + bo_ref[...]
    y = jnp.maximum(y, 0.0)
    out_ref[...] = y + concat


@functools.partial(jax.jit, static_argnames=())
def _run(x2, ln1_g, ln1_b, Wq, bq, Wk, bk, Wv, bv, ln2_g, ln2_b, Wo, bo):
    grid = (_S // _T,)
    vec = pl.BlockSpec((1, _D), lambda i: (0, 0))
    mat = pl.BlockSpec((_D, _D), lambda i: (0, 0))
    out = pl.pallas_call(
        _body,
        grid=grid,
        in_specs=[
            pl.BlockSpec((_T, _D), lambda i: (i, 0)),
            pl.BlockSpec((_W, _D),
                         lambda i: (jnp.maximum(i * (_T // _W) - 1, 0), 0)),
            vec, vec, mat, vec, mat, vec, mat, vec, vec, vec, mat, vec,
        ],
        out_specs=pl.BlockSpec((_T, _D), lambda i: (i, 0)),
        out_shape=jax.ShapeDtypeStruct((_S, _D), jnp.float32),
        compiler_params=pltpu.CompilerParams(
            dimension_semantics=("arbitrary",),
        ),
    )(x2, x2, ln1_g, ln1_b, Wq, bq, Wk, bk, Wv, bv, ln2_g, ln2_b, Wo, bo)
    return out


def kernel(inputs, ln1_g, ln1_b, Wq, bq, Wk, bk, Wv, bv, ln2_g, ln2_b, Wo, bo, mask):
    del mask
    x2 = inputs.reshape(_S, _D)
    r = lambda a: a.reshape(1, _D)
    scale = 1.0 / (_DEPTH ** 0.5)
    out = _run(x2, r(ln1_g), r(ln1_b), Wq * scale, r(bq * scale), Wk, r(bk),
               Wv, r(bv), r(ln2_g), r(ln2_b), Wo, r(bo))
    return out.reshape(1, _S, _D)


# speedup vs baseline: 1.0728x; 1.0390x over previous
import functools

import jax
import jax.numpy as jnp
from jax.experimental import pallas as pl
from jax.experimental.pallas import tpu as pltpu

_S = 4096
_D = 768
_H = 12
_DEPTH = 64
_W = 8
_T = 512


def _layer_norm_in(x, g, b, eps=1e-3):
    mu = jnp.mean(x, axis=-1, keepdims=True)
    xc = x - mu
    var = jnp.mean(xc * xc, axis=-1, keepdims=True)
    return g * xc * jax.lax.rsqrt(var + eps) + b


def _body(xb_ref, halo_ref, g1_ref, b1_ref, wq_ref, bq_ref, wk_ref, bk_ref,
          wv_ref, bv_ref, g2_ref, b2_ref, wo_ref, bo_ref, out_ref):
    i = pl.program_id(0)
    xfull = jnp.concatenate([halo_ref[...], xb_ref[...]], axis=0)
    xl = _layer_norm_in(xfull, g1_ref[...], b1_ref[...])

    xq = xl[_W:, :]
    q = (jnp.dot(xq, wq_ref[...], preferred_element_type=jnp.float32)
         + bq_ref[...]) * (1.0 / (_DEPTH ** 0.5))
    k = jnp.dot(xl, wk_ref[...], preferred_element_type=jnp.float32) + bk_ref[...]
    v = jnp.dot(xl, wv_ref[...], preferred_element_type=jnp.float32) + bv_ref[...]

    r_idx = jax.lax.broadcasted_iota(jnp.int32, (_T, _T + _W), 0)
    m_idx = jax.lax.broadcasted_iota(jnp.int32, (_T, _T + _W), 1)
    diff = r_idx + _W - m_idx
    valid = (diff >= 0) & (diff < _W) & (m_idx + i * _T >= _W)

    chunks = []
    for h in range(_H):
        sl = slice(h * _DEPTH, (h + 1) * _DEPTH)
        lg = jax.lax.dot_general(q[:, sl], k[:, sl], (((1,), (1,)), ((), ())),
                                 preferred_element_type=jnp.float32)
        e = jnp.where(valid, jnp.exp(lg), 0.0)
        den = jnp.sum(e, axis=1, keepdims=True)
        num = jnp.dot(e, v[:, sl], preferred_element_type=jnp.float32)
        chunks.append(num / den)
    att = jnp.concatenate(chunks, axis=1)

    concat = xq + att
    y = _layer_norm_in(concat, g2_ref[...], b2_ref[...])
    y = jnp.dot(y, wo_ref[...], preferred_element_type=jnp.float32) + bo_ref[...]
    y = jnp.maximum(y, 0.0)
    out_ref[...] = y + concat


@functools.partial(jax.jit, static_argnames=())
def _run(x2, ln1_g, ln1_b, Wq, bq, Wk, bk, Wv, bv, ln2_g, ln2_b, Wo, bo):
    grid = (_S // _T,)
    vec = pl.BlockSpec((1, _D), lambda i: (0, 0))
    mat = pl.BlockSpec((_D, _D), lambda i: (0, 0))
    out = pl.pallas_call(
        _body,
        grid=grid,
        in_specs=[
            pl.BlockSpec((_T, _D), lambda i: (i, 0)),
            pl.BlockSpec((_W, _D),
                         lambda i: (jnp.maximum(i * (_T // _W) - 1, 0), 0)),
            vec, vec, mat, vec, mat, vec, mat, vec, vec, vec, mat, vec,
        ],
        out_specs=pl.BlockSpec((_T, _D), lambda i: (i, 0)),
        out_shape=jax.ShapeDtypeStruct((_S, _D), jnp.float32),
        compiler_params=pltpu.CompilerParams(
            dimension_semantics=("arbitrary",),
        ),
    )(x2, x2, ln1_g, ln1_b, Wq, bq, Wk, bk, Wv, bv, ln2_g, ln2_b, Wo, bo)
    return out


def kernel(inputs, ln1_g, ln1_b, Wq, bq, Wk, bk, Wv, bv, ln2_g, ln2_b, Wo, bo, mask):
    del mask
    x2 = inputs.reshape(_S, _D)
    r = lambda a: a.reshape(1, _D)
    out = _run(x2, r(ln1_g), r(ln1_b), Wq, r(bq), Wk, r(bk),
               Wv, r(bv), r(ln2_g), r(ln2_b), Wo, r(bo))
    return out.reshape(1, _S, _D)
